# Initial kernel scaffold; baseline (speedup 1.0000x reference)
import jax
import jax.numpy as jnp
from jax.experimental import pallas as pl


def kernel(h_drug, h_prot, conv1_w_dd, conv1_w_pp, conv1_w_d2p, conv1_w_p2d, conv1_w_ppp, conv2_w_dd, conv2_w_pp, conv2_w_d2p, conv2_w_p2d, conv2_w_ppp, dpi_src, dpi_dst, ppi_src, ppi_dst):
    raise NotImplementedError("write your pallas kernel here")


# speedup vs baseline: 1.9303x; 1.9303x over previous
import functools

import jax
import jax.numpy as jnp
from jax import lax
from jax.experimental import pallas as pl
from jax.experimental.pallas import tpu as pltpu
from jax.experimental.pallas import tpu_sc as plsc

_NS = 16
_B = 128
_C = 128
_NCHUNK = 4
_CW = _C // _NCHUNK


def _fill_zeros(ref, nrow):
    z = jnp.zeros((16,), jnp.float32)

    def body(i, _):
        ref[i, pl.ds(0, 16)] = z
        ref[i, pl.ds(16, 16)] = z
        return 0

    lax.fori_loop(0, nrow, body, 0)


@functools.lru_cache(maxsize=None)
def _make_seg_sum(n_dst, n_edge):
    nb_full = n_edge // _B
    rem = n_edge - nb_full * _B
    nb_per_tile = (nb_full + _NS - 1) // _NS
    zrow = 256
    rpt = n_dst // _NS
    nz_full, z_rem = rpt // zrow, rpt % zrow

    mesh = plsc.VectorSubcoreMesh(core_axis_name="c", subcore_axis_name="s", num_cores=2, num_subcores=_NS)

    scratch = [
        pltpu.VMEM((_B,), jnp.int32),
        pltpu.VMEM((_B,), jnp.int32),
        pltpu.VMEM((_B, _CW), jnp.float32),
        pltpu.VMEM((zrow, _CW), jnp.float32),
        pltpu.VMEM_SHARED((n_dst, _CW), jnp.float32),
        pltpu.SemaphoreType.DMA,
    ]
    if rem:
        scratch += [
            pltpu.VMEM((rem,), jnp.int32),
            pltpu.VMEM((rem,), jnp.int32),
            pltpu.VMEM((rem, _CW), jnp.float32),
        ]

    @functools.partial(
        pl.kernel,
        out_type=jax.ShapeDtypeStruct((n_dst, _C), jnp.float32),
        mesh=mesh,
        scratch_types=scratch,
        compiler_params=pltpu.CompilerParams(use_tc_tiling_on_sc=False),
    )
    def seg_kernel(table, src, dst, out, gidx_v, didx_v, rows_v, zbuf_v,
                   acc_sh, sem, *rem_bufs):
        core = lax.axis_index("c")
        sid = lax.axis_index("s")

        _fill_zeros(zbuf_v, zrow)

        def do_block(chunk, base, gi, di, rows):
            pltpu.sync_copy(src.at[pl.ds(base, gi.shape[0])], gi)
            pltpu.sync_copy(dst.at[pl.ds(base, di.shape[0])], di)

            def tbody(k, _):
                o = pl.multiple_of(k * 16, 16)
                gi[pl.ds(o, 16)] = gi[pl.ds(o, 16)] * _NCHUNK + chunk
                return 0

            lax.fori_loop(0, gi.shape[0] // 16, tbody, 0)
            pltpu.async_copy(table.at[gi], rows, sem).wait()
            pltpu.sync_copy(rows, acc_sh.at[di], add=True)

        for chunk in range(_NCHUNK):
            @pl.when(core == chunk // 2)
            def _():
                r0 = sid * rpt
                for j in range(nz_full):
                    pltpu.sync_copy(zbuf_v, acc_sh.at[pl.ds(r0 + j * zrow, zrow)])
                if z_rem:
                    pltpu.sync_copy(zbuf_v.at[pl.ds(0, z_rem)],
                                    acc_sh.at[pl.ds(r0 + nz_full * zrow, z_rem)])
                plsc.subcore_barrier()

                def ebody(i, _):
                    b = i * _NS + sid

                    @pl.when(b < nb_full)
                    def _():
                        do_block(chunk, b * _B, gidx_v, didx_v, rows_v)
                    return 0

                lax.fori_loop(0, nb_per_tile, ebody, 0)
                if rem:
                    @pl.when(sid == 0)
                    def _():
                        do_block(chunk, nb_full * _B, *rem_bufs)
                plsc.subcore_barrier()
                pltpu.sync_copy(
                    acc_sh.at[pl.ds(r0, rpt)],
                    out.at[pl.ds(r0, rpt), pl.ds(chunk * _CW, _CW)])
                plsc.subcore_barrier()

    return seg_kernel


@functools.lru_cache(maxsize=None)
def _make_counts(n_drug, n_prot, e_dpi, e_ppi):
    mesh = plsc.VectorSubcoreMesh(core_axis_name="c", subcore_axis_name="s", num_cores=2, num_subcores=_NS)

    nb_dpi = e_dpi // _B
    rem_dpi = e_dpi - nb_dpi * _B
    nb_ppi = e_ppi // _B
    rem_ppi = e_ppi - nb_ppi * _B

    zlen = 8192
    scratch = [
        pltpu.VMEM((_B,), jnp.int32),
        pltpu.VMEM((_B,), jnp.int32),
        pltpu.VMEM((_B,), jnp.float32),
        pltpu.VMEM((zlen,), jnp.float32),
        pltpu.VMEM_SHARED((n_drug,), jnp.float32),
        pltpu.VMEM_SHARED((n_prot,), jnp.float32),
    ]
    if rem_dpi:
        scratch += [pltpu.VMEM((rem_dpi,), jnp.int32),
                    pltpu.VMEM((rem_dpi,), jnp.int32),
                    pltpu.VMEM((rem_dpi,), jnp.float32)]
    if rem_ppi:
        scratch += [pltpu.VMEM((rem_ppi,), jnp.int32),
                    pltpu.VMEM((rem_ppi,), jnp.float32)]

    @functools.partial(
        pl.kernel,
        out_type=[jax.ShapeDtypeStruct((n_drug,), jnp.float32),
                  jax.ShapeDtypeStruct((n_prot,), jnp.float32),
                  jax.ShapeDtypeStruct((n_prot,), jnp.float32)],
        mesh=mesh,
        scratch_types=scratch,
        compiler_params=pltpu.CompilerParams(use_tc_tiling_on_sc=False),
    )
    def cnt_kernel(dpi_src, dpi_dst, ppi_dst, cnt_d, cnt_p1, cnt_p2,
                   ia_v, ib_v, ones_v, zbuf_v, accs_sh, accp_sh, *rem_bufs):
        core = lax.axis_index("c")
        sid = lax.axis_index("s")

        one = jnp.full((16,), 1.0, jnp.float32)
        for k in range(_B // 16):
            ones_v[pl.ds(k * 16, 16)] = one
        z = jnp.zeros((16,), jnp.float32)

        def zfill(i, _):
            o = pl.multiple_of(i * 16, 16)
            zbuf_v[pl.ds(o, 16)] = z
            return 0

        lax.fori_loop(0, zlen // 16, zfill, 0)

        def zero_acc(acc, n):
            full, zr = n // zlen, n % zlen
            for j in range(full):
                pltpu.sync_copy(zbuf_v, acc.at[pl.ds(j * zlen, zlen)])
            if zr:
                pltpu.sync_copy(zbuf_v.at[pl.ds(0, zr)],
                                acc.at[pl.ds(full * zlen, zr)])

        def scan(idx_hbm, acc, nb_full, two_level=None):
            def ebody(i, _):
                b = i * _NS + sid

                @pl.when(b < nb_full)
                def _():
                    pltpu.sync_copy(idx_hbm.at[pl.ds(b * _B, _B)], ia_v)
                    pltpu.sync_copy(ones_v, acc.at[ia_v], add=True)
                    if two_level is not None:
                        oidx, oacc = two_level
                        pltpu.sync_copy(oidx.at[pl.ds(b * _B, _B)], ib_v)
                        pltpu.sync_copy(ones_v, oacc.at[ib_v], add=True)
                return 0

            lax.fori_loop(0, (nb_full + _NS - 1) // _NS, ebody, 0)

        def scan_rem(idx_hbm, acc, nb_full, rem, ri, ro, two=None):
            @pl.when(sid == 0)
            def _():
                for k in range(rem // 16):
                    ro[pl.ds(k * 16, 16)] = one
                pltpu.sync_copy(idx_hbm.at[pl.ds(nb_full * _B, rem)], ri)
                pltpu.sync_copy(ro, acc.at[ri], add=True)
                if two is not None:
                    oidx, oacc, rj = two
                    pltpu.sync_copy(oidx.at[pl.ds(nb_full * _B, rem)], rj)
                    pltpu.sync_copy(ro, oacc.at[rj], add=True)

        @pl.when(core == 0)
        def _():
            @pl.when(sid == 0)
            def _():
                zero_acc(accs_sh, n_drug)

            @pl.when(sid == 1)
            def _():
                zero_acc(accp_sh, n_prot)
            plsc.subcore_barrier()
            scan(dpi_src, accs_sh, nb_dpi, two_level=(dpi_dst, accp_sh))
            if rem_dpi:
                scan_rem(dpi_src, accs_sh, nb_dpi, rem_dpi,
                         rem_bufs[0], rem_bufs[2],
                         two=(dpi_dst, accp_sh, rem_bufs[1]))
            plsc.subcore_barrier()

            @pl.when(sid == 0)
            def _():
                pltpu.sync_copy(accs_sh, cnt_d)
                pltpu.sync_copy(accp_sh, cnt_p1)

        @pl.when(core == 1)
        def _():
            @pl.when(sid == 0)
            def _():
                zero_acc(accp_sh, n_prot)
            plsc.subcore_barrier()
            scan(ppi_dst, accp_sh, nb_ppi)
            if rem_ppi:
                nr = 3 if rem_dpi else 0
                scan_rem(ppi_dst, accp_sh, nb_ppi, rem_ppi,
                         rem_bufs[nr], rem_bufs[nr + 1])
            plsc.subcore_barrier()

            @pl.when(sid == 0)
            def _():
                pltpu.sync_copy(accp_sh, cnt_p2)

    return cnt_kernel


_BN = 2000


def _dot(a, b):
    return lax.dot_general(a, b, (((1,), (0,)), ((), ())),
                           precision=lax.Precision.HIGHEST,
                           preferred_element_type=jnp.float32)


@functools.lru_cache(maxsize=None)
def _make_drug_tc(n, residual):
    grid = (n // _BN,)

    def body(*refs):
        if residual:
            hd, agg, cnt, wdd, wp2d, res, out = refs
        else:
            hd, agg, cnt, wdd, wp2d, out = refs
        rec = 1.0 / jnp.maximum(cnt[...], 1.0)
        acc = _dot(hd[...], wdd[...]) + _dot(agg[...] * rec, wp2d[...])
        if residual:
            acc = acc + res[...]
        out[...] = jnp.maximum(acc, 0.0)

    row = pl.BlockSpec((_BN, _C), lambda i: (i, 0))
    col = pl.BlockSpec((_BN, 1), lambda i: (i, 0))
    w = pl.BlockSpec((_C, _C), lambda i: (0, 0))
    in_specs = [row, row, col, w, w] + ([row] if residual else [])
    return pl.pallas_call(
        body, grid=grid, in_specs=in_specs, out_specs=row,
        out_shape=jax.ShapeDtypeStruct((n, _C), jnp.float32))


@functools.lru_cache(maxsize=None)
def _make_prot_tc(n, residual):
    grid = (n // _BN,)

    def body(*refs):
        if residual:
            hp, a1, c1, a2, c2, wpp, wd2p, wppp, res, out = refs
        else:
            hp, a1, c1, a2, c2, wpp, wd2p, wppp, out = refs
        r1 = 1.0 / jnp.maximum(c1[...], 1.0)
        r2 = 1.0 / jnp.maximum(c2[...], 1.0)
        acc = (_dot(hp[...], wpp[...]) + _dot(a1[...] * r1, wd2p[...])
               + _dot(a2[...] * r2, wppp[...]))
        if residual:
            acc = acc + res[...]
        out[...] = jnp.maximum(acc, 0.0)

    row = pl.BlockSpec((_BN, _C), lambda i: (i, 0))
    col = pl.BlockSpec((_BN, 1), lambda i: (i, 0))
    w = pl.BlockSpec((_C, _C), lambda i: (0, 0))
    in_specs = [row, row, col, row, col, w, w, w] + ([row] if residual else [])
    return pl.pallas_call(
        body, grid=grid, in_specs=in_specs, out_specs=row,
        out_shape=jax.ShapeDtypeStruct((n, _C), jnp.float32))


def kernel(h_drug, h_prot,
           conv1_w_dd, conv1_w_pp, conv1_w_d2p, conv1_w_p2d, conv1_w_ppp,
           conv2_w_dd, conv2_w_pp, conv2_w_d2p, conv2_w_p2d, conv2_w_ppp,
           dpi_src, dpi_dst, ppi_src, ppi_dst):
    n_drug, c = h_drug.shape
    n_prot = h_prot.shape[0]
    e_dpi = dpi_src.shape[0]
    e_ppi = ppi_src.shape[0]

    cnt_k = _make_counts(n_drug, n_prot, e_dpi, e_ppi)
    cnt_d, cnt_p1, cnt_p2 = cnt_k(dpi_src, dpi_dst, ppi_dst)
    cnt_d = cnt_d.reshape(n_drug, 1)
    cnt_p1 = cnt_p1.reshape(n_prot, 1)
    cnt_p2 = cnt_p2.reshape(n_prot, 1)

    seg_d = _make_seg_sum(n_drug, e_dpi)
    seg_p1 = _make_seg_sum(n_prot, e_dpi)
    seg_p2 = _make_seg_sum(n_prot, e_ppi)

    hd, hp = h_drug, h_prot
    for wdd, wpp, wd2p, wp2d, wppp, res in (
            (conv1_w_dd, conv1_w_pp, conv1_w_d2p, conv1_w_p2d, conv1_w_ppp, None),
            (conv2_w_dd, conv2_w_pp, conv2_w_d2p, conv2_w_p2d, conv2_w_ppp,
             (h_drug, h_prot))):
        hdf = hd.reshape(n_drug * _NCHUNK, _CW)
        hpf = hp.reshape(n_prot * _NCHUNK, _CW)
        a_d = seg_d(hpf, dpi_dst, dpi_src)
        a_p1 = seg_p1(hdf, dpi_src, dpi_dst)
        a_p2 = seg_p2(hpf, ppi_src, ppi_dst)
        if res is None:
            hd = _make_drug_tc(n_drug, False)(hd, a_d, cnt_d, wdd, wp2d)
            hp = _make_prot_tc(n_prot, False)(hp, a_p1, cnt_p1, a_p2, cnt_p2,
                                              wpp, wd2p, wppp)
        else:
            hd = _make_drug_tc(n_drug, True)(hd, a_d, cnt_d, wdd, wp2d, res[0])
            hp = _make_prot_tc(n_prot, True)(hp, a_p1, cnt_p1, a_p2, cnt_p2,
                                             wpp, wd2p, wppp, res[1])
    return hd, hp


# speedup vs baseline: 2.4233x; 1.2554x over previous
import functools

import jax
import jax.numpy as jnp
from jax import lax
from jax.experimental import pallas as pl
from jax.experimental.pallas import tpu as pltpu
from jax.experimental.pallas import tpu_sc as plsc

_NS = 16
_B = 128
_C = 128
_NCHUNK = 4
_CW = _C // _NCHUNK


def _fill_zeros(ref, nrow):
    z = jnp.zeros((16,), jnp.float32)

    def body(i, _):
        ref[i, pl.ds(0, 16)] = z
        ref[i, pl.ds(16, 16)] = z
        return 0

    lax.fori_loop(0, nrow, body, 0)


_NB = 4


@functools.lru_cache(maxsize=None)
def _make_seg_sum(n_dst, nbt):
    rows_pt = nbt // _NS
    ngrp = rows_pt // _NB
    n_acc = n_dst + 16
    zrow = 128
    rpt_z = n_acc // _NS
    nz_full, z_rem = rpt_z // zrow, rpt_z % zrow
    rpt_o = n_dst // _NS

    mesh = plsc.VectorSubcoreMesh(core_axis_name="c", subcore_axis_name="s", num_cores=2, num_subcores=_NS)

    scratch = (
        [pltpu.VMEM((_B,), jnp.int32) for _ in range(_NB)]
        + [pltpu.VMEM((_B,), jnp.int32) for _ in range(_NB)]
        + [pltpu.VMEM((_B, _CW), jnp.float32) for _ in range(_NB)]
        + [
            pltpu.VMEM((zrow, _CW), jnp.float32),
            pltpu.VMEM_SHARED((n_acc, _CW), jnp.float32),
            pltpu.SemaphoreType.DMA,
        ]
    )

    @functools.partial(
        pl.kernel,
        out_type=jax.ShapeDtypeStruct((n_dst, _C), jnp.float32),
        mesh=mesh,
        scratch_types=scratch,
        compiler_params=pltpu.CompilerParams(use_tc_tiling_on_sc=False),
    )
    def seg_kernel(table, src, dst, out, *bufs):
        gi = bufs[0:_NB]
        di = bufs[_NB:2 * _NB]
        ring = bufs[2 * _NB:3 * _NB]
        zbuf_v, acc_sh, sem = bufs[3 * _NB:]
        core = lax.axis_index("c")
        sid = lax.axis_index("s")

        _fill_zeros(zbuf_v, zrow)

        r0i = sid * rows_pt
        c0 = core * 2

        def fire(j, b, addv):
            row = r0i + j
            pltpu.sync_copy(src.at[row], gi[b])
            pltpu.sync_copy(dst.at[row], di[b])
            for g in range(_B // 16):
                o = g * 16
                gi[b][pl.ds(o, 16)] = gi[b][pl.ds(o, 16)] * _NCHUNK + addv
            pltpu.async_copy(table.at[gi[b]], ring[b], sem)

        def drain_scatter(b):
            pltpu.make_async_copy(table.at[gi[b]], ring[b], sem).wait()
            pltpu.sync_copy(ring[b], acc_sh.at[di[b]], add=True)

        for cc in range(2):
            addv = c0 + cc
            r0z = sid * rpt_z
            for j in range(nz_full):
                pltpu.sync_copy(zbuf_v, acc_sh.at[pl.ds(r0z + j * zrow, zrow)])
            if z_rem:
                pltpu.sync_copy(zbuf_v.at[pl.ds(0, z_rem)],
                                acc_sh.at[pl.ds(r0z + nz_full * zrow, z_rem)])
            plsc.subcore_barrier()

            for b in range(_NB):
                fire(b, b, addv)

            def gbody(i, _):
                base = i * _NB
                for b in range(_NB):
                    drain_scatter(b)
                    fire(base + _NB + b, b, addv)
                return 0

            lax.fori_loop(0, ngrp - 1, gbody, 0)
            for b in range(_NB):
                drain_scatter(b)
            plsc.subcore_barrier()
            r0o = sid * rpt_o
            pltpu.sync_copy(
                acc_sh.at[pl.ds(r0o, rpt_o)],
                out.at[pl.ds(r0o, rpt_o), pl.ds((c0 + cc) * _CW, _CW)])
            plsc.subcore_barrier()

    return seg_kernel


@functools.lru_cache(maxsize=None)
def _make_counts(n_drug, n_prot, e_dpi, e_ppi):
    mesh = plsc.VectorSubcoreMesh(core_axis_name="c", subcore_axis_name="s", num_cores=2, num_subcores=_NS)

    nb_dpi = e_dpi // _B
    rem_dpi = e_dpi - nb_dpi * _B
    nb_ppi = e_ppi // _B
    rem_ppi = e_ppi - nb_ppi * _B

    zlen = 8192
    scratch = [
        pltpu.VMEM((_B,), jnp.int32),
        pltpu.VMEM((_B,), jnp.int32),
        pltpu.VMEM((_B,), jnp.float32),
        pltpu.VMEM((zlen,), jnp.float32),
        pltpu.VMEM_SHARED((n_drug,), jnp.float32),
        pltpu.VMEM_SHARED((n_prot,), jnp.float32),
    ]
    if rem_dpi:
        scratch += [pltpu.VMEM((rem_dpi,), jnp.int32),
                    pltpu.VMEM((rem_dpi,), jnp.int32),
                    pltpu.VMEM((rem_dpi,), jnp.float32)]
    if rem_ppi:
        scratch += [pltpu.VMEM((rem_ppi,), jnp.int32),
                    pltpu.VMEM((rem_ppi,), jnp.float32)]

    @functools.partial(
        pl.kernel,
        out_type=[jax.ShapeDtypeStruct((n_drug,), jnp.float32),
                  jax.ShapeDtypeStruct((n_prot,), jnp.float32),
                  jax.ShapeDtypeStruct((n_prot,), jnp.float32)],
        mesh=mesh,
        scratch_types=scratch,
        compiler_params=pltpu.CompilerParams(use_tc_tiling_on_sc=False),
    )
    def cnt_kernel(dpi_src, dpi_dst, ppi_dst, cnt_d, cnt_p1, cnt_p2,
                   ia_v, ib_v, ones_v, zbuf_v, accs_sh, accp_sh, *rem_bufs):
        core = lax.axis_index("c")
        sid = lax.axis_index("s")

        one = jnp.full((16,), 1.0, jnp.float32)
        for k in range(_B // 16):
            ones_v[pl.ds(k * 16, 16)] = one
        z = jnp.zeros((16,), jnp.float32)

        def zfill(i, _):
            o = pl.multiple_of(i * 16, 16)
            zbuf_v[pl.ds(o, 16)] = z
            return 0

        lax.fori_loop(0, zlen // 16, zfill, 0)

        def zero_acc(acc, n):
            full, zr = n // zlen, n % zlen
            for j in range(full):
                pltpu.sync_copy(zbuf_v, acc.at[pl.ds(j * zlen, zlen)])
            if zr:
                pltpu.sync_copy(zbuf_v.at[pl.ds(0, zr)],
                                acc.at[pl.ds(full * zlen, zr)])

        def scan(idx_hbm, acc, nb_full, two_level=None):
            def ebody(i, _):
                b = i * _NS + sid

                @pl.when(b < nb_full)
                def _():
                    pltpu.sync_copy(idx_hbm.at[pl.ds(b * _B, _B)], ia_v)
                    pltpu.sync_copy(ones_v, acc.at[ia_v], add=True)
                    if two_level is not None:
                        oidx, oacc = two_level
                        pltpu.sync_copy(oidx.at[pl.ds(b * _B, _B)], ib_v)
                        pltpu.sync_copy(ones_v, oacc.at[ib_v], add=True)
                return 0

            lax.fori_loop(0, (nb_full + _NS - 1) // _NS, ebody, 0)

        def scan_rem(idx_hbm, acc, nb_full, rem, ri, ro, two=None):
            @pl.when(sid == 0)
            def _():
                for k in range(rem // 16):
                    ro[pl.ds(k * 16, 16)] = one
                pltpu.sync_copy(idx_hbm.at[pl.ds(nb_full * _B, rem)], ri)
                pltpu.sync_copy(ro, acc.at[ri], add=True)
                if two is not None:
                    oidx, oacc, rj = two
                    pltpu.sync_copy(oidx.at[pl.ds(nb_full * _B, rem)], rj)
                    pltpu.sync_copy(ro, oacc.at[rj], add=True)

        @pl.when(core == 0)
        def _():
            @pl.when(sid == 0)
            def _():
                zero_acc(accs_sh, n_drug)

            @pl.when(sid == 1)
            def _():
                zero_acc(accp_sh, n_prot)
            plsc.subcore_barrier()
            scan(dpi_src, accs_sh, nb_dpi, two_level=(dpi_dst, accp_sh))
            if rem_dpi:
                scan_rem(dpi_src, accs_sh, nb_dpi, rem_dpi,
                         rem_bufs[0], rem_bufs[2],
                         two=(dpi_dst, accp_sh, rem_bufs[1]))
            plsc.subcore_barrier()

            @pl.when(sid == 0)
            def _():
                pltpu.sync_copy(accs_sh, cnt_d)
                pltpu.sync_copy(accp_sh, cnt_p1)

        @pl.when(core == 1)
        def _():
            @pl.when(sid == 0)
            def _():
                zero_acc(accp_sh, n_prot)
            plsc.subcore_barrier()
            scan(ppi_dst, accp_sh, nb_ppi)
            if rem_ppi:
                nr = 3 if rem_dpi else 0
                scan_rem(ppi_dst, accp_sh, nb_ppi, rem_ppi,
                         rem_bufs[nr], rem_bufs[nr + 1])
            plsc.subcore_barrier()

            @pl.when(sid == 0)
            def _():
                pltpu.sync_copy(accp_sh, cnt_p2)

    return cnt_kernel


_BN = 2000


def _dot(a, b):
    return lax.dot_general(a, b, (((1,), (0,)), ((), ())),
                           precision=lax.Precision.HIGHEST,
                           preferred_element_type=jnp.float32)


@functools.lru_cache(maxsize=None)
def _make_drug_tc(n, residual):
    grid = (n // _BN,)

    def body(*refs):
        if residual:
            hd, agg, cnt, wdd, wp2d, res, out = refs
        else:
            hd, agg, cnt, wdd, wp2d, out = refs
        rec = 1.0 / jnp.maximum(cnt[...], 1.0)
        acc = _dot(hd[...], wdd[...]) + _dot(agg[...] * rec, wp2d[...])
        if residual:
            acc = acc + res[...]
        out[...] = jnp.maximum(acc, 0.0)

    row = pl.BlockSpec((_BN, _C), lambda i: (i, 0))
    col = pl.BlockSpec((_BN, 1), lambda i: (i, 0))
    w = pl.BlockSpec((_C, _C), lambda i: (0, 0))
    in_specs = [row, row, col, w, w] + ([row] if residual else [])
    return pl.pallas_call(
        body, grid=grid, in_specs=in_specs, out_specs=row,
        out_shape=jax.ShapeDtypeStruct((n, _C), jnp.float32))


@functools.lru_cache(maxsize=None)
def _make_prot_tc(n, residual):
    grid = (n // _BN,)

    def body(*refs):
        if residual:
            hp, a1, c1, a2, c2, wpp, wd2p, wppp, res, out = refs
        else:
            hp, a1, c1, a2, c2, wpp, wd2p, wppp, out = refs
        r1 = 1.0 / jnp.maximum(c1[...], 1.0)
        r2 = 1.0 / jnp.maximum(c2[...], 1.0)
        acc = (_dot(hp[...], wpp[...]) + _dot(a1[...] * r1, wd2p[...])
               + _dot(a2[...] * r2, wppp[...]))
        if residual:
            acc = acc + res[...]
        out[...] = jnp.maximum(acc, 0.0)

    row = pl.BlockSpec((_BN, _C), lambda i: (i, 0))
    col = pl.BlockSpec((_BN, 1), lambda i: (i, 0))
    w = pl.BlockSpec((_C, _C), lambda i: (0, 0))
    in_specs = [row, row, col, row, col, w, w, w] + ([row] if residual else [])
    return pl.pallas_call(
        body, grid=grid, in_specs=in_specs, out_specs=row,
        out_shape=jax.ShapeDtypeStruct((n, _C), jnp.float32))


def kernel(h_drug, h_prot,
           conv1_w_dd, conv1_w_pp, conv1_w_d2p, conv1_w_p2d, conv1_w_ppp,
           conv2_w_dd, conv2_w_pp, conv2_w_d2p, conv2_w_p2d, conv2_w_ppp,
           dpi_src, dpi_dst, ppi_src, ppi_dst):
    n_drug, c = h_drug.shape
    n_prot = h_prot.shape[0]
    e_dpi = dpi_src.shape[0]
    e_ppi = ppi_src.shape[0]

    cnt_k = _make_counts(n_drug, n_prot, e_dpi, e_ppi)
    cnt_d, cnt_p1, cnt_p2 = cnt_k(dpi_src, dpi_dst, ppi_dst)
    cnt_d = cnt_d.reshape(n_drug, 1)
    cnt_p1 = cnt_p1.reshape(n_prot, 1)
    cnt_p2 = cnt_p2.reshape(n_prot, 1)

    def _pad2d(idx, fill):
        nbt = -(-idx.shape[0] // _B)
        nbt = -(-nbt // (_NS * _NB)) * (_NS * _NB)
        pad = nbt * _B - idx.shape[0]
        if pad:
            idx = jnp.concatenate([idx, jnp.full((pad,), fill, jnp.int32)])
        return idx.reshape(nbt, _B), nbt

    dpi_src_g, nbt_dpi = _pad2d(dpi_src, 0)
    dpi_src_s, _ = _pad2d(dpi_src, n_drug)
    dpi_dst_g, _ = _pad2d(dpi_dst, 0)
    dpi_dst_s, _ = _pad2d(dpi_dst, n_prot)
    ppi_src_g, nbt_ppi = _pad2d(ppi_src, 0)
    ppi_dst_s, _ = _pad2d(ppi_dst, n_prot)

    seg_d = _make_seg_sum(n_drug, nbt_dpi)
    seg_p1 = _make_seg_sum(n_prot, nbt_dpi)
    seg_p2 = _make_seg_sum(n_prot, nbt_ppi)

    hd, hp = h_drug, h_prot
    for wdd, wpp, wd2p, wp2d, wppp, res in (
            (conv1_w_dd, conv1_w_pp, conv1_w_d2p, conv1_w_p2d, conv1_w_ppp, None),
            (conv2_w_dd, conv2_w_pp, conv2_w_d2p, conv2_w_p2d, conv2_w_ppp,
             (h_drug, h_prot))):
        hdf = hd.reshape(n_drug * _NCHUNK, _CW)
        hpf = hp.reshape(n_prot * _NCHUNK, _CW)
        a_d = seg_d(hpf, dpi_dst_g, dpi_src_s)
        a_p1 = seg_p1(hdf, dpi_src_g, dpi_dst_s)
        a_p2 = seg_p2(hpf, ppi_src_g, ppi_dst_s)
        if res is None:
            hd = _make_drug_tc(n_drug, False)(hd, a_d, cnt_d, wdd, wp2d)
            hp = _make_prot_tc(n_prot, False)(hp, a_p1, cnt_p1, a_p2, cnt_p2,
                                              wpp, wd2p, wppp)
        else:
            hd = _make_drug_tc(n_drug, True)(hd, a_d, cnt_d, wdd, wp2d, res[0])
            hp = _make_prot_tc(n_prot, True)(hp, a_p1, cnt_p1, a_p2, cnt_p2,
                                             wpp, wd2p, wppp, res[1])
    return hd, hp


# speedup vs baseline: 2.5431x; 1.0494x over previous
import functools

import jax
import jax.numpy as jnp
from jax import lax
from jax.experimental import pallas as pl
from jax.experimental.pallas import tpu as pltpu
from jax.experimental.pallas import tpu_sc as plsc

_NS = 16
_B = 128
_C = 128


def _fill_zeros(ref, nrow, ncol):
    z = jnp.zeros((16,), jnp.float32)

    def body(i, _):
        for o in range(0, ncol, 16):
            ref[i, pl.ds(o, 16)] = z
        return 0

    lax.fori_loop(0, nrow, body, 0)


_NB = 4


@functools.lru_cache(maxsize=None)
def _make_seg_sum(n_dst, nbt, nchunk):
    cw = _C // nchunk
    cpc = nchunk // 2
    rows_pt = nbt // _NS
    ngrp = rows_pt // _NB
    n_acc = n_dst + 16
    zrow = 128
    rpt_z = n_acc // _NS
    nz_full, z_rem = rpt_z // zrow, rpt_z % zrow
    rpt_o = n_dst // _NS

    mesh = plsc.VectorSubcoreMesh(core_axis_name="c", subcore_axis_name="s", num_cores=2, num_subcores=_NS)

    scratch = (
        [pltpu.VMEM((_B,), jnp.int32) for _ in range(_NB)]
        + [pltpu.VMEM((_B,), jnp.int32) for _ in range(_NB)]
        + [pltpu.VMEM((_B, cw), jnp.float32) for _ in range(_NB)]
        + [
            pltpu.VMEM((zrow, cw), jnp.float32),
            pltpu.VMEM_SHARED((n_acc, cw), jnp.float32),
            pltpu.SemaphoreType.DMA,
        ]
    )

    @functools.partial(
        pl.kernel,
        out_type=jax.ShapeDtypeStruct((n_dst, _C), jnp.float32),
        mesh=mesh,
        scratch_types=scratch,
        compiler_params=pltpu.CompilerParams(use_tc_tiling_on_sc=False),
    )
    def seg_kernel(table, src, dst, out, *bufs):
        gi = bufs[0:_NB]
        di = bufs[_NB:2 * _NB]
        ring = bufs[2 * _NB:3 * _NB]
        zbuf_v, acc_sh, sem = bufs[3 * _NB:]
        core = lax.axis_index("c")
        sid = lax.axis_index("s")

        _fill_zeros(zbuf_v, zrow, cw)

        r0i = sid * rows_pt
        c0 = core * cpc

        def fire(j, b, addv):
            row = r0i + j
            pltpu.sync_copy(src.at[row], gi[b])
            pltpu.sync_copy(dst.at[row], di[b])
            for g in range(_B // 16):
                o = g * 16
                gi[b][pl.ds(o, 16)] = gi[b][pl.ds(o, 16)] * nchunk + addv
            pltpu.async_copy(table.at[gi[b]], ring[b], sem)

        def drain_scatter(b):
            pltpu.make_async_copy(table.at[gi[b]], ring[b], sem).wait()
            pltpu.sync_copy(ring[b], acc_sh.at[di[b]], add=True)

        for cc in range(cpc):
            addv = c0 + cc
            r0z = sid * rpt_z
            for j in range(nz_full):
                pltpu.sync_copy(zbuf_v, acc_sh.at[pl.ds(r0z + j * zrow, zrow)])
            if z_rem:
                pltpu.sync_copy(zbuf_v.at[pl.ds(0, z_rem)],
                                acc_sh.at[pl.ds(r0z + nz_full * zrow, z_rem)])
            plsc.subcore_barrier()

            for b in range(_NB):
                fire(b, b, addv)

            def gbody(i, _):
                base = i * _NB
                for b in range(_NB):
                    drain_scatter(b)
                    fire(base + _NB + b, b, addv)
                return 0

            lax.fori_loop(0, ngrp - 1, gbody, 0)
            for b in range(_NB):
                drain_scatter(b)
            plsc.subcore_barrier()
            r0o = sid * rpt_o
            pltpu.sync_copy(
                acc_sh.at[pl.ds(r0o, rpt_o)],
                out.at[pl.ds(r0o, rpt_o), pl.ds(addv * cw, cw)])
            plsc.subcore_barrier()

    return seg_kernel


@functools.lru_cache(maxsize=None)
def _make_counts(n_drug, n_prot, e_dpi, e_ppi):
    mesh = plsc.VectorSubcoreMesh(core_axis_name="c", subcore_axis_name="s", num_cores=2, num_subcores=_NS)

    nb_dpi = e_dpi // _B
    rem_dpi = e_dpi - nb_dpi * _B
    nb_ppi = e_ppi // _B
    rem_ppi = e_ppi - nb_ppi * _B

    zlen = 8192
    scratch = [
        pltpu.VMEM((_B,), jnp.int32),
        pltpu.VMEM((_B,), jnp.int32),
        pltpu.VMEM((_B,), jnp.float32),
        pltpu.VMEM((zlen,), jnp.float32),
        pltpu.VMEM_SHARED((n_drug,), jnp.float32),
        pltpu.VMEM_SHARED((n_prot,), jnp.float32),
    ]
    if rem_dpi:
        scratch += [pltpu.VMEM((rem_dpi,), jnp.int32),
                    pltpu.VMEM((rem_dpi,), jnp.int32),
                    pltpu.VMEM((rem_dpi,), jnp.float32)]
    if rem_ppi:
        scratch += [pltpu.VMEM((rem_ppi,), jnp.int32),
                    pltpu.VMEM((rem_ppi,), jnp.float32)]

    @functools.partial(
        pl.kernel,
        out_type=[jax.ShapeDtypeStruct((n_drug,), jnp.float32),
                  jax.ShapeDtypeStruct((n_prot,), jnp.float32),
                  jax.ShapeDtypeStruct((n_prot,), jnp.float32)],
        mesh=mesh,
        scratch_types=scratch,
        compiler_params=pltpu.CompilerParams(use_tc_tiling_on_sc=False),
    )
    def cnt_kernel(dpi_src, dpi_dst, ppi_dst, cnt_d, cnt_p1, cnt_p2,
                   ia_v, ib_v, ones_v, zbuf_v, accs_sh, accp_sh, *rem_bufs):
        core = lax.axis_index("c")
        sid = lax.axis_index("s")

        one = jnp.full((16,), 1.0, jnp.float32)
        for k in range(_B // 16):
            ones_v[pl.ds(k * 16, 16)] = one
        z = jnp.zeros((16,), jnp.float32)

        def zfill(i, _):
            o = pl.multiple_of(i * 16, 16)
            zbuf_v[pl.ds(o, 16)] = z
            return 0

        lax.fori_loop(0, zlen // 16, zfill, 0)

        def zero_acc(acc, n):
            full, zr = n // zlen, n % zlen
            for j in range(full):
                pltpu.sync_copy(zbuf_v, acc.at[pl.ds(j * zlen, zlen)])
            if zr:
                pltpu.sync_copy(zbuf_v.at[pl.ds(0, zr)],
                                acc.at[pl.ds(full * zlen, zr)])

        def scan(idx_hbm, acc, nb_full, two_level=None):
            def ebody(i, _):
                b = i * _NS + sid

                @pl.when(b < nb_full)
                def _():
                    pltpu.sync_copy(idx_hbm.at[pl.ds(b * _B, _B)], ia_v)
                    pltpu.sync_copy(ones_v, acc.at[ia_v], add=True)
                    if two_level is not None:
                        oidx, oacc = two_level
                        pltpu.sync_copy(oidx.at[pl.ds(b * _B, _B)], ib_v)
                        pltpu.sync_copy(ones_v, oacc.at[ib_v], add=True)
                return 0

            lax.fori_loop(0, (nb_full + _NS - 1) // _NS, ebody, 0)

        def scan_rem(idx_hbm, acc, nb_full, rem, ri, ro, two=None):
            @pl.when(sid == 0)
            def _():
                for k in range(rem // 16):
                    ro[pl.ds(k * 16, 16)] = one
                pltpu.sync_copy(idx_hbm.at[pl.ds(nb_full * _B, rem)], ri)
                pltpu.sync_copy(ro, acc.at[ri], add=True)
                if two is not None:
                    oidx, oacc, rj = two
                    pltpu.sync_copy(oidx.at[pl.ds(nb_full * _B, rem)], rj)
                    pltpu.sync_copy(ro, oacc.at[rj], add=True)

        @pl.when(core == 0)
        def _():
            @pl.when(sid == 0)
            def _():
                zero_acc(accs_sh, n_drug)

            @pl.when(sid == 1)
            def _():
                zero_acc(accp_sh, n_prot)
            plsc.subcore_barrier()
            scan(dpi_src, accs_sh, nb_dpi, two_level=(dpi_dst, accp_sh))
            if rem_dpi:
                scan_rem(dpi_src, accs_sh, nb_dpi, rem_dpi,
                         rem_bufs[0], rem_bufs[2],
                         two=(dpi_dst, accp_sh, rem_bufs[1]))
            plsc.subcore_barrier()

            @pl.when(sid == 0)
            def _():
                pltpu.sync_copy(accs_sh, cnt_d)
                pltpu.sync_copy(accp_sh, cnt_p1)

        @pl.when(core == 1)
        def _():
            @pl.when(sid == 0)
            def _():
                zero_acc(accp_sh, n_prot)
            plsc.subcore_barrier()
            scan(ppi_dst, accp_sh, nb_ppi)
            if rem_ppi:
                nr = 3 if rem_dpi else 0
                scan_rem(ppi_dst, accp_sh, nb_ppi, rem_ppi,
                         rem_bufs[nr], rem_bufs[nr + 1])
            plsc.subcore_barrier()

            @pl.when(sid == 0)
            def _():
                pltpu.sync_copy(accp_sh, cnt_p2)

    return cnt_kernel


_BN = 2000


def _dot(a, b):
    return lax.dot_general(a, b, (((1,), (0,)), ((), ())),
                           precision=lax.Precision.HIGHEST,
                           preferred_element_type=jnp.float32)


@functools.lru_cache(maxsize=None)
def _make_drug_tc(n, residual):
    grid = (n // _BN,)

    def body(*refs):
        if residual:
            hd, agg, cnt, wdd, wp2d, res, out = refs
        else:
            hd, agg, cnt, wdd, wp2d, out = refs
        rec = 1.0 / jnp.maximum(cnt[...], 1.0)
        acc = _dot(hd[...], wdd[...]) + _dot(agg[...] * rec, wp2d[...])
        if residual:
            acc = acc + res[...]
        out[...] = jnp.maximum(acc, 0.0)

    row = pl.BlockSpec((_BN, _C), lambda i: (i, 0))
    col = pl.BlockSpec((_BN, 1), lambda i: (i, 0))
    w = pl.BlockSpec((_C, _C), lambda i: (0, 0))
    in_specs = [row, row, col, w, w] + ([row] if residual else [])
    return pl.pallas_call(
        body, grid=grid, in_specs=in_specs, out_specs=row,
        out_shape=jax.ShapeDtypeStruct((n, _C), jnp.float32))


@functools.lru_cache(maxsize=None)
def _make_prot_tc(n, residual):
    grid = (n // _BN,)

    def body(*refs):
        if residual:
            hp, a1, c1, a2, c2, wpp, wd2p, wppp, res, out = refs
        else:
            hp, a1, c1, a2, c2, wpp, wd2p, wppp, out = refs
        r1 = 1.0 / jnp.maximum(c1[...], 1.0)
        r2 = 1.0 / jnp.maximum(c2[...], 1.0)
        acc = (_dot(hp[...], wpp[...]) + _dot(a1[...] * r1, wd2p[...])
               + _dot(a2[...] * r2, wppp[...]))
        if residual:
            acc = acc + res[...]
        out[...] = jnp.maximum(acc, 0.0)

    row = pl.BlockSpec((_BN, _C), lambda i: (i, 0))
    col = pl.BlockSpec((_BN, 1), lambda i: (i, 0))
    w = pl.BlockSpec((_C, _C), lambda i: (0, 0))
    in_specs = [row, row, col, row, col, w, w, w] + ([row] if residual else [])
    return pl.pallas_call(
        body, grid=grid, in_specs=in_specs, out_specs=row,
        out_shape=jax.ShapeDtypeStruct((n, _C), jnp.float32))


def kernel(h_drug, h_prot,
           conv1_w_dd, conv1_w_pp, conv1_w_d2p, conv1_w_p2d, conv1_w_ppp,
           conv2_w_dd, conv2_w_pp, conv2_w_d2p, conv2_w_p2d, conv2_w_ppp,
           dpi_src, dpi_dst, ppi_src, ppi_dst):
    n_drug, c = h_drug.shape
    n_prot = h_prot.shape[0]
    e_dpi = dpi_src.shape[0]
    e_ppi = ppi_src.shape[0]

    cnt_k = _make_counts(n_drug, n_prot, e_dpi, e_ppi)
    cnt_d, cnt_p1, cnt_p2 = cnt_k(dpi_src, dpi_dst, ppi_dst)
    cnt_d = cnt_d.reshape(n_drug, 1)
    cnt_p1 = cnt_p1.reshape(n_prot, 1)
    cnt_p2 = cnt_p2.reshape(n_prot, 1)

    def _pad2d(idx, fill):
        nbt = -(-idx.shape[0] // _B)
        nbt = -(-nbt // (_NS * _NB)) * (_NS * _NB)
        pad = nbt * _B - idx.shape[0]
        if pad:
            idx = jnp.concatenate([idx, jnp.full((pad,), fill, jnp.int32)])
        return idx.reshape(nbt, _B), nbt

    dpi_src_g, nbt_dpi = _pad2d(dpi_src, 0)
    dpi_src_s, _ = _pad2d(dpi_src, n_drug)
    dpi_dst_g, _ = _pad2d(dpi_dst, 0)
    dpi_dst_s, _ = _pad2d(dpi_dst, n_prot)
    ppi_src_g, nbt_ppi = _pad2d(ppi_src, 0)
    ppi_dst_s, _ = _pad2d(ppi_dst, n_prot)

    def _nchunk_for(n):
        return 2 if (n + 16) * 64 + 700_000 <= 2_000_000 else 4

    nc_d = _nchunk_for(n_drug)
    nc_p = _nchunk_for(n_prot)
    seg_d = _make_seg_sum(n_drug, nbt_dpi, nc_d)
    seg_p1 = _make_seg_sum(n_prot, nbt_dpi, nc_p)
    seg_p2 = _make_seg_sum(n_prot, nbt_ppi, nc_p)

    hd, hp = h_drug, h_prot
    for wdd, wpp, wd2p, wp2d, wppp, res in (
            (conv1_w_dd, conv1_w_pp, conv1_w_d2p, conv1_w_p2d, conv1_w_ppp, None),
            (conv2_w_dd, conv2_w_pp, conv2_w_d2p, conv2_w_p2d, conv2_w_ppp,
             (h_drug, h_prot))):
        a_d = seg_d(hp.reshape(n_prot * nc_d, _C // nc_d),
                    dpi_dst_g, dpi_src_s)
        a_p1 = seg_p1(hd.reshape(n_drug * nc_p, _C // nc_p),
                      dpi_src_g, dpi_dst_s)
        a_p2 = seg_p2(hp.reshape(n_prot * nc_p, _C // nc_p),
                      ppi_src_g, ppi_dst_s)
        if res is None:
            hd = _make_drug_tc(n_drug, False)(hd, a_d, cnt_d, wdd, wp2d)
            hp = _make_prot_tc(n_prot, False)(hp, a_p1, cnt_p1, a_p2, cnt_p2,
                                              wpp, wd2p, wppp)
        else:
            hd = _make_drug_tc(n_drug, True)(hd, a_d, cnt_d, wdd, wp2d, res[0])
            hp = _make_prot_tc(n_prot, True)(hp, a_p1, cnt_p1, a_p2, cnt_p2,
                                             wpp, wd2p, wppp, res[1])
    return hd, hp
